# baseline (device time: 86686 ns/iter reference)
import jax
import jax.numpy as jnp
from jax import lax
from jax.experimental import pallas as pl
from jax.experimental.pallas import tpu as pltpu

N_DEV = 4
M = 1024
D = 1024


def kernel(partial, gamma):
    x = partial.reshape(N_DEV * M, D)
    g = gamma.reshape(1, D)

    def body(x_ref, g_ref, out_ref, comm_ref, send_sems, recv_sems):
        my_x = lax.axis_index("x")
        my_y = lax.axis_index("y")
        my_z = lax.axis_index("z")
        left = (my_z + N_DEV - 1) % N_DEV
        right = (my_z + 1) % N_DEV

        barrier_sem = pltpu.get_barrier_semaphore()
        for nbr in (left, right):
            pl.semaphore_signal(
                barrier_sem, inc=1,
                device_id=(my_x, my_y, nbr),
                device_id_type=pl.DeviceIdType.MESH,
            )
        pl.semaphore_wait(barrier_sem, 2)

        c0 = (my_z + N_DEV - 1) % N_DEV
        comm_ref[0] = x_ref[pl.ds(c0 * M, M), :].astype(jnp.bfloat16)

        for h in range(N_DEV - 1):
            rdma = pltpu.make_async_remote_copy(
                src_ref=comm_ref.at[h],
                dst_ref=comm_ref.at[h + 1],
                send_sem=send_sems.at[h],
                recv_sem=recv_sems.at[h],
                device_id=(my_x, my_y, right),
                device_id_type=pl.DeviceIdType.MESH,
            )
            rdma.start()
            rdma.wait()

            c_recv = (my_z + N_DEV - h - 2) % N_DEV
            acc = comm_ref[h + 1].astype(jnp.float32) + x_ref[
                pl.ds(c_recv * M, M), :
            ]
            if h < N_DEV - 2:
                comm_ref[h + 1] = acc.astype(jnp.bfloat16)
            else:
                rms = jnp.sqrt(
                    jnp.mean(acc * acc, axis=-1, keepdims=True) + 1e-6
                )
                out_ref[...] = acc / rms * g_ref[...]

    return pl.pallas_call(
        body,
        out_shape=jax.ShapeDtypeStruct((M, D), jnp.float32),
        in_specs=[
            pl.BlockSpec(memory_space=pltpu.VMEM),
            pl.BlockSpec(memory_space=pltpu.VMEM),
        ],
        out_specs=pl.BlockSpec(memory_space=pltpu.VMEM),
        scratch_shapes=[
            pltpu.VMEM((N_DEV, M, D), jnp.bfloat16),
            pltpu.SemaphoreType.DMA((N_DEV - 1,)),
            pltpu.SemaphoreType.DMA((N_DEV - 1,)),
        ],
        compiler_params=pltpu.CompilerParams(collective_id=0),
    )(x, g)


# device time: 49926 ns/iter; 1.7363x vs baseline; 1.7363x over previous
import jax
import jax.numpy as jnp
from jax import lax
from jax.experimental import pallas as pl
from jax.experimental.pallas import tpu as pltpu

Z = 4
M = 1024
MS = 256
D = 1024

_MESH = pl.DeviceIdType.MESH


def kernel(partial, gamma):
    x = partial.reshape(Z * M, D)
    g = gamma.reshape(1, D)

    def body(x_ref, g_ref, out_ref,
             rr_buf, lr_buf, acc_ref, snd_ref, bx_ref, by_ref, bd_ref,
             rr_send, rr_recv, lr_send, lr_recv, pb_send, pb_recv):
        my_x = lax.axis_index("x")
        my_y = lax.axis_index("y")
        my_z = lax.axis_index("z")
        r = 2 * my_x + my_y
        zp = jnp.minimum(my_z + 1, Z - 1)
        zm = jnp.maximum(my_z - 1, 0)

        def own(c):
            return x_ref[pl.ds(c * M + r * MS, MS), :]

        bsem = pltpu.get_barrier_semaphore()
        pl.semaphore_signal(bsem, inc=1, device_id=(1 - my_x, my_y, my_z),
                            device_id_type=_MESH)
        pl.semaphore_signal(bsem, inc=1, device_id=(my_x, 1 - my_y, my_z),
                            device_id_type=_MESH)

        @pl.when(my_z > 0)
        def _():
            pl.semaphore_signal(bsem, inc=1, device_id=(my_x, my_y, zm),
                                device_id_type=_MESH)

        @pl.when(my_z < Z - 1)
        def _():
            pl.semaphore_signal(bsem, inc=1, device_id=(my_x, my_y, zp),
                                device_id_type=_MESH)

        pl.semaphore_wait(bsem, 3)

        @pl.when((my_z > 0) & (my_z < Z - 1))
        def _():
            pl.semaphore_wait(bsem, 1)

        def rr_desc(c):
            return pltpu.make_async_remote_copy(
                src_ref=rr_buf.at[c], dst_ref=rr_buf.at[c],
                send_sem=rr_send.at[c], recv_sem=rr_recv.at[c],
                device_id=(my_x, my_y, zp), device_id_type=_MESH)

        def lr_desc(c):
            return pltpu.make_async_remote_copy(
                src_ref=lr_buf.at[c], dst_ref=lr_buf.at[c],
                send_sem=lr_send.at[c], recv_sem=lr_recv.at[c],
                device_id=(my_x, my_y, zm), device_id_type=_MESH)

        def rw_step(c):
            @pl.when((my_z >= 1) & (my_z < c))
            def _():
                rr_desc(c).wait_recv()

            @pl.when(my_z < c)
            def _():
                prev = jnp.where(my_z >= 1,
                                 rr_buf[c].astype(jnp.float32), 0.0)
                rr_buf[c] = (prev + own(c)).astype(jnp.bfloat16)
                rr_desc(c).start()

        def lw_step(c):
            @pl.when((my_z <= Z - 2) & (my_z > c))
            def _():
                lr_desc(c).wait_recv()

            @pl.when(my_z > c)
            def _():
                prev = jnp.where(my_z <= Z - 2,
                                 lr_buf[c].astype(jnp.float32), 0.0)
                lr_buf[c] = (prev + own(c)).astype(jnp.bfloat16)
                lr_desc(c).start()

        @pl.when(my_z % 2 == 1)
        def _():
            rw_step(3); lw_step(0); rw_step(2); lw_step(1)
            rw_step(1); lw_step(2)

        @pl.when(my_z % 2 == 0)
        def _():
            lw_step(0); rw_step(3); lw_step(1); rw_step(2)
            lw_step(2); rw_step(1)

        for c in range(Z):
            if c >= 1:
                @pl.when(my_z == c)
                def _(c=c):
                    rr_desc(c).wait_recv()
            if c <= Z - 2:
                @pl.when(my_z == c)
                def _(c=c):
                    lr_desc(c).wait_recv()

            @pl.when(my_z == c)
            def _(c=c):
                acc = own(c)
                if c >= 1:
                    acc = acc + rr_buf[c].astype(jnp.float32)
                if c <= Z - 2:
                    acc = acc + lr_buf[c].astype(jnp.float32)
                acc_ref[...] = acc

        y = acc_ref[...]
        rms = jnp.sqrt(jnp.mean(y * y, axis=-1, keepdims=True) + 1e-6)
        normed = y / rms * g_ref[...]
        out_ref[pl.ds(r * MS, MS), :] = normed
        snd_ref[...] = normed.astype(jnp.bfloat16)

        s1x = pltpu.make_async_remote_copy(
            src_ref=snd_ref, dst_ref=bx_ref,
            send_sem=pb_send.at[0], recv_sem=pb_recv.at[0],
            device_id=(1 - my_x, my_y, my_z), device_id_type=_MESH)
        s1y = pltpu.make_async_remote_copy(
            src_ref=snd_ref, dst_ref=by_ref,
            send_sem=pb_send.at[1], recv_sem=pb_recv.at[1],
            device_id=(my_x, 1 - my_y, my_z), device_id_type=_MESH)
        s2 = pltpu.make_async_remote_copy(
            src_ref=bx_ref, dst_ref=bd_ref,
            send_sem=pb_send.at[2], recv_sem=pb_recv.at[2],
            device_id=(my_x, 1 - my_y, my_z), device_id_type=_MESH)
        s1x.start()
        s1y.start()
        s1x.wait_recv()
        s2.start()
        r_x = 2 * (1 - my_x) + my_y
        out_ref[pl.ds(r_x * MS, MS), :] = bx_ref[...].astype(jnp.float32)
        s1y.wait_recv()
        r_y = 2 * my_x + (1 - my_y)
        out_ref[pl.ds(r_y * MS, MS), :] = by_ref[...].astype(jnp.float32)
        s2.wait_recv()
        r_d = 2 * (1 - my_x) + (1 - my_y)
        out_ref[pl.ds(r_d * MS, MS), :] = bd_ref[...].astype(jnp.float32)

        s1x.wait_send()
        s1y.wait_send()
        s2.wait_send()
        for c in range(1, Z):
            @pl.when(my_z < c)
            def _(c=c):
                rr_desc(c).wait_send()
        for c in range(Z - 1):
            @pl.when(my_z > c)
            def _(c=c):
                lr_desc(c).wait_send()

    return pl.pallas_call(
        body,
        out_shape=jax.ShapeDtypeStruct((M, D), jnp.float32),
        in_specs=[
            pl.BlockSpec(memory_space=pltpu.VMEM),
            pl.BlockSpec(memory_space=pltpu.VMEM),
        ],
        out_specs=pl.BlockSpec(memory_space=pltpu.VMEM),
        scratch_shapes=[
            pltpu.VMEM((Z, MS, D), jnp.bfloat16),
            pltpu.VMEM((Z, MS, D), jnp.bfloat16),
            pltpu.VMEM((MS, D), jnp.float32),
            pltpu.VMEM((MS, D), jnp.bfloat16),
            pltpu.VMEM((MS, D), jnp.bfloat16),
            pltpu.VMEM((MS, D), jnp.bfloat16),
            pltpu.VMEM((MS, D), jnp.bfloat16),
            pltpu.SemaphoreType.DMA((Z,)),
            pltpu.SemaphoreType.DMA((Z,)),
            pltpu.SemaphoreType.DMA((Z,)),
            pltpu.SemaphoreType.DMA((Z,)),
            pltpu.SemaphoreType.DMA((3,)),
            pltpu.SemaphoreType.DMA((3,)),
        ],
        compiler_params=pltpu.CompilerParams(collective_id=0),
    )(x, g)
